# Initial kernel scaffold; baseline (speedup 1.0000x reference)
import jax
import jax.numpy as jnp
from jax.experimental import pallas as pl


def kernel(tokens, organ_priors, router_W, router_b, W1, b1, W2, b2):
    raise NotImplementedError("write your pallas kernel here")


# speedup vs baseline: 6.8038x; 6.8038x over previous
import functools
import math

import jax
import jax.numpy as jnp
from jax.experimental import pallas as pl
from jax.experimental.pallas import tpu as pltpu

_INTERPRET = False

B = 1
T = 2048
D_MODEL = 768
ORGAN_DIM = 64
N_EXPERTS = 64
D_FF = 2048
CAPACITY = int(math.ceil(T * B / N_EXPERTS * 1.25))
SLOTS = 48
NFF = 2
FFB = D_FF // NFF
NEG_INF = float("-inf")


def _router_kernel(flat_ref, prior_ref, rw_ref, rb_ref,
                   probs_ref, ent_ref, sel_ref, selw_ref,
                   s_ref, wf_ref):
    flat = flat_ref[:]
    prior = prior_ref[:]
    inp = jnp.concatenate([flat, prior], axis=1)
    logits = jax.lax.dot_general(
        inp, rw_ref[:], (((1,), (1,)), ((), ())),
        preferred_element_type=jnp.float32) + rb_ref[:]
    m = jnp.max(logits, axis=1, keepdims=True)
    ex = jnp.exp(logits - m)
    probs = ex / jnp.sum(ex, axis=1, keepdims=True)
    probs_ref[:] = probs
    ent_ref[:] = -jnp.sum(probs * jnp.log(probs + 1e-12), axis=1,
                          keepdims=True)

    lane = jax.lax.broadcasted_iota(jnp.int32, (T, N_EXPERTS), 1)
    m1 = jnp.max(probs, axis=1, keepdims=True)
    i1 = jnp.min(jnp.where(probs == m1, lane, N_EXPERTS), axis=1,
                 keepdims=True)
    p2 = jnp.where(lane == i1, -1.0, probs)
    m2 = jnp.max(p2, axis=1, keepdims=True)
    i2 = jnp.min(jnp.where(p2 == m2, lane, N_EXPERTS), axis=1,
                 keepdims=True)
    denom = m1 + m2 + 1e-12
    hit1 = lane == i1
    hit2 = lane == i2
    wf_ref[:] = (jnp.where(hit1, m1 / denom, 0.0)
                 + jnp.where(hit2, m2 / denom, 0.0))
    s_ref[:] = jnp.where(hit1 | hit2, probs, NEG_INF)

    sel_ref[:] = jnp.full((SLOTS, N_EXPERTS), T, jnp.int32)
    selw_ref[:] = jnp.zeros((SLOTS, N_EXPERTS), jnp.float32)

    rows = jax.lax.broadcasted_iota(jnp.int32, (T, N_EXPERTS), 0)

    def body(k, _):
        sv = s_ref[:]
        mk = jnp.max(sv, axis=0, keepdims=True)
        cand = jnp.where(sv == mk, rows, T)
        idx = jnp.min(cand, axis=0, keepdims=True)
        valid = mk > -1e37
        picked = rows == idx
        wsel = jnp.sum(jnp.where(picked, wf_ref[:], 0.0), axis=0,
                       keepdims=True)
        sel_ref[pl.ds(k, 1), :] = jnp.where(valid, idx, T)
        selw_ref[pl.ds(k, 1), :] = jnp.where(valid, wsel, 0.0)
        s_ref[:] = jnp.where(picked, NEG_INF, sv)
        return 0

    jax.lax.fori_loop(0, CAPACITY, body, 0)


def _expert_kernel(flat_ref, sel_ref, selw_ref, b1_ref, b2_ref,
                   w1_ref, w2_ref, out_ref, xsel_ref, osel_ref, proc_ref):
    e = pl.program_id(0)
    f = pl.program_id(1)

    lane = jax.lax.broadcasted_iota(jnp.int32, (SLOTS, N_EXPERTS), 1)
    sel_col = jnp.sum(jnp.where(lane == e, sel_ref[:], 0), axis=1,
                      keepdims=True)
    w_col = jnp.sum(jnp.where(lane == e, selw_ref[:], 0.0), axis=1,
                    keepdims=True)
    d0 = jax.lax.broadcasted_iota(jnp.int32, (SLOTS, SLOTS), 0)
    d1 = jax.lax.broadcasted_iota(jnp.int32, (SLOTS, SLOTS), 1)
    diag = d0 == d1
    sel_row = jnp.sum(jnp.where(diag, sel_col, 0), axis=0, keepdims=True)
    w_row = jnp.sum(jnp.where(diag, w_col, 0.0), axis=0, keepdims=True)

    @pl.when(f == 0)
    def _():
        pt = (jax.lax.broadcasted_iota(jnp.int32, (SLOTS, T), 1)
              == sel_col).astype(jnp.float32)
        xsel_ref[:] = jnp.dot(pt, flat_ref[:],
                              preferred_element_type=jnp.float32)
        osel_ref[:] = jnp.zeros((SLOTS, D_MODEL), jnp.float32)

    a = jnp.dot(xsel_ref[:], w1_ref[0],
                preferred_element_type=jnp.float32) + b1_ref[0]
    h = 0.5 * a * (1.0 + jax.lax.erf(a * (1.0 / math.sqrt(2.0))))
    osel_ref[:] += jnp.dot(h, w2_ref[0],
                           preferred_element_type=jnp.float32)

    @pl.when(f == NFF - 1)
    def _():
        tok = jax.lax.broadcasted_iota(jnp.int32, (T, SLOTS), 0)
        p01 = (tok == sel_row).astype(jnp.float32)
        pw = p01 * w_row

        @pl.when(e == 0)
        def _():
            out_ref[:] = jnp.zeros((T, D_MODEL), jnp.float32)
            proc_ref[:] = jnp.zeros((T, 1), jnp.float32)

        osel = osel_ref[:] + b2_ref[0]
        out_ref[:] += jnp.dot(pw, osel, preferred_element_type=jnp.float32)
        proc_ref[:] += jnp.sum(p01, axis=1, keepdims=True)

        @pl.when(e == N_EXPERTS - 1)
        def _():
            out_ref[:] = jnp.where(proc_ref[:] > 0.0, out_ref[:],
                                   flat_ref[:])


@jax.jit
def kernel(tokens, organ_priors, router_W, router_b, W1, b1, W2, b2):
    flat = tokens.reshape(T, D_MODEL)
    prior = organ_priors.reshape(T, ORGAN_DIM)
    rb = router_b.reshape(1, N_EXPERTS)

    probs, ent, sel, selw = pl.pallas_call(
        _router_kernel,
        out_shape=[
            jax.ShapeDtypeStruct((T, N_EXPERTS), jnp.float32),
            jax.ShapeDtypeStruct((T, 1), jnp.float32),
            jax.ShapeDtypeStruct((SLOTS, N_EXPERTS), jnp.int32),
            jax.ShapeDtypeStruct((SLOTS, N_EXPERTS), jnp.float32),
        ],
        scratch_shapes=[
            pltpu.VMEM((T, N_EXPERTS), jnp.float32),
            pltpu.VMEM((T, N_EXPERTS), jnp.float32),
        ],
        interpret=_INTERPRET,
    )(flat, prior, router_W, rb)

    b1r = b1.reshape(N_EXPERTS, 1, D_FF)
    b2r = b2.reshape(N_EXPERTS, 1, D_MODEL)

    out = pl.pallas_call(
        _expert_kernel,
        grid=(N_EXPERTS, NFF),
        in_specs=[
            pl.BlockSpec((T, D_MODEL), lambda e, f: (0, 0)),
            pl.BlockSpec((SLOTS, N_EXPERTS), lambda e, f: (0, 0)),
            pl.BlockSpec((SLOTS, N_EXPERTS), lambda e, f: (0, 0)),
            pl.BlockSpec((1, 1, FFB), lambda e, f: (e, 0, f)),
            pl.BlockSpec((1, 1, D_MODEL), lambda e, f: (e, 0, 0)),
            pl.BlockSpec((1, D_MODEL, FFB), lambda e, f: (e, 0, f)),
            pl.BlockSpec((1, FFB, D_MODEL), lambda e, f: (e, f, 0)),
        ],
        out_specs=pl.BlockSpec((T, D_MODEL), lambda e, f: (0, 0)),
        out_shape=jax.ShapeDtypeStruct((T, D_MODEL), jnp.float32),
        scratch_shapes=[
            pltpu.VMEM((SLOTS, D_MODEL), jnp.float32),
            pltpu.VMEM((SLOTS, D_MODEL), jnp.float32),
            pltpu.VMEM((T, 1), jnp.float32),
        ],
        compiler_params=pltpu.CompilerParams(
            dimension_semantics=("arbitrary", "arbitrary"),
        ),
        interpret=_INTERPRET,
    )(flat, sel, selw, b1r, b2r, W1, W2)

    return (out.reshape(B, T, D_MODEL),
            probs.reshape(B, T, N_EXPERTS),
            ent.reshape(B, T))


# speedup vs baseline: 6.8553x; 1.0076x over previous
import functools
import math

import jax
import jax.numpy as jnp
from jax.experimental import pallas as pl
from jax.experimental.pallas import tpu as pltpu

_INTERPRET = False

B = 1
T = 2048
D_MODEL = 768
ORGAN_DIM = 64
N_EXPERTS = 64
D_FF = 2048
CAPACITY = int(math.ceil(T * B / N_EXPERTS * 1.25))
SLOTS = 40
NFF = 2
FFB = D_FF // NFF
NEG_INF = float("-inf")


def _bf(x):
    return x.astype(jnp.bfloat16)


def _moe_kernel(flat_ref, prior_ref, rw_ref, rb_ref, b1_ref, b2_ref,
                w1_ref, w2_ref,
                out_ref, probs_ref, ent_ref,
                s_ref, wf_ref, sel_ref, selw_ref,
                xsel_ref, osel_ref, proc_ref):
    e = pl.program_id(0)
    f = pl.program_id(1)

    @pl.when((e == 0) & (f == 0))
    def _router():
        flat = flat_ref[:]
        inp = jnp.concatenate([flat, prior_ref[:]], axis=1)
        logits = jax.lax.dot_general(
            inp, rw_ref[:], (((1,), (1,)), ((), ())),
            preferred_element_type=jnp.float32) + rb_ref[:]
        m = jnp.max(logits, axis=1, keepdims=True)
        ex = jnp.exp(logits - m)
        probs = ex / jnp.sum(ex, axis=1, keepdims=True)
        probs_ref[:] = probs
        ent_ref[:] = -jnp.sum(probs * jnp.log(probs + 1e-12), axis=1,
                              keepdims=True)

        lane = jax.lax.broadcasted_iota(jnp.int32, (T, N_EXPERTS), 1)
        m1 = jnp.max(probs, axis=1, keepdims=True)
        i1 = jnp.min(jnp.where(probs == m1, lane, N_EXPERTS), axis=1,
                     keepdims=True)
        p2 = jnp.where(lane == i1, -1.0, probs)
        m2 = jnp.max(p2, axis=1, keepdims=True)
        i2 = jnp.min(jnp.where(p2 == m2, lane, N_EXPERTS), axis=1,
                     keepdims=True)
        denom = m1 + m2 + 1e-12
        hit1 = lane == i1
        hit2 = lane == i2
        wf_ref[:] = (jnp.where(hit1, m1 / denom, 0.0)
                     + jnp.where(hit2, m2 / denom, 0.0))
        s_ref[:] = jnp.where(hit1 | hit2, probs, NEG_INF)

        rows = jax.lax.broadcasted_iota(jnp.int32, (T, N_EXPERTS), 0)

        def body(k, _):
            sv = s_ref[:]
            mk = jnp.max(sv, axis=0, keepdims=True)
            cand = jnp.where(sv == mk, rows, T)
            idx = jnp.min(cand, axis=0, keepdims=True)
            valid = mk > -1e37
            picked = rows == idx
            wsel = jnp.sum(jnp.where(picked, wf_ref[:], 0.0), axis=0,
                           keepdims=True)
            sel_ref[pl.ds(k, 1), :] = jnp.where(valid, idx, T)
            selw_ref[pl.ds(k, 1), :] = jnp.where(valid, wsel, 0.0)
            s_ref[:] = jnp.where(picked, NEG_INF, sv)
            return 0

        jax.lax.fori_loop(0, CAPACITY, body, 0)

    lane = jax.lax.broadcasted_iota(jnp.int32, (SLOTS, N_EXPERTS), 1)
    sel_col = jnp.sum(jnp.where(lane == e, sel_ref[:], 0), axis=1,
                      keepdims=True)

    @pl.when(f == 0)
    def _gather():
        pt = (jax.lax.broadcasted_iota(jnp.int32, (SLOTS, T), 1)
              == sel_col).astype(jnp.bfloat16)
        xsel_ref[:] = jnp.dot(pt, _bf(flat_ref[:]),
                              preferred_element_type=jnp.float32)
        osel_ref[:] = jnp.zeros((SLOTS, D_MODEL), jnp.float32)

    a = jnp.dot(_bf(xsel_ref[:]), _bf(w1_ref[0]),
                preferred_element_type=jnp.float32) + b1_ref[0]
    h = 0.5 * a * (1.0 + jax.lax.erf(a * (1.0 / math.sqrt(2.0))))
    osel_ref[:] += jnp.dot(_bf(h), _bf(w2_ref[0]),
                           preferred_element_type=jnp.float32)

    @pl.when(f == NFF - 1)
    def _scatter():
        w_col = jnp.sum(jnp.where(lane == e, selw_ref[:], 0.0), axis=1,
                        keepdims=True)
        d0 = jax.lax.broadcasted_iota(jnp.int32, (SLOTS, SLOTS), 0)
        d1 = jax.lax.broadcasted_iota(jnp.int32, (SLOTS, SLOTS), 1)
        diag = d0 == d1
        sel_row = jnp.sum(jnp.where(diag, sel_col, 0), axis=0,
                          keepdims=True)
        w_row = jnp.sum(jnp.where(diag, w_col, 0.0), axis=0,
                        keepdims=True)
        tok = jax.lax.broadcasted_iota(jnp.int32, (T, SLOTS), 0)
        p01 = tok == sel_row
        pw = jnp.where(p01, w_row, 0.0).astype(jnp.bfloat16)

        @pl.when(e == 0)
        def _():
            out_ref[:] = jnp.zeros((T, D_MODEL), jnp.float32)
            proc_ref[:] = jnp.zeros((T, 1), jnp.float32)

        osel = osel_ref[:] + b2_ref[0]
        out_ref[:] += jnp.dot(pw, _bf(osel),
                              preferred_element_type=jnp.float32)
        proc_ref[:] += jnp.sum(p01.astype(jnp.float32), axis=1,
                               keepdims=True)

        @pl.when(e == N_EXPERTS - 1)
        def _():
            out_ref[:] = jnp.where(proc_ref[:] > 0.0, out_ref[:],
                                   flat_ref[:])


@jax.jit
def kernel(tokens, organ_priors, router_W, router_b, W1, b1, W2, b2):
    flat = tokens.reshape(T, D_MODEL)
    prior = organ_priors.reshape(T, ORGAN_DIM)
    rb = router_b.reshape(1, N_EXPERTS)
    b1r = b1.reshape(N_EXPERTS, 1, D_FF)
    b2r = b2.reshape(N_EXPERTS, 1, D_MODEL)

    out, probs, ent = pl.pallas_call(
        _moe_kernel,
        grid=(N_EXPERTS, NFF),
        in_specs=[
            pl.BlockSpec((T, D_MODEL), lambda e, f: (0, 0)),
            pl.BlockSpec((T, ORGAN_DIM), lambda e, f: (0, 0)),
            pl.BlockSpec((N_EXPERTS, D_MODEL + ORGAN_DIM),
                         lambda e, f: (0, 0)),
            pl.BlockSpec((1, N_EXPERTS), lambda e, f: (0, 0)),
            pl.BlockSpec((1, 1, FFB), lambda e, f: (e, 0, f)),
            pl.BlockSpec((1, 1, D_MODEL), lambda e, f: (e, 0, 0)),
            pl.BlockSpec((1, D_MODEL, FFB), lambda e, f: (e, 0, f)),
            pl.BlockSpec((1, FFB, D_MODEL), lambda e, f: (e, f, 0)),
        ],
        out_specs=[
            pl.BlockSpec((T, D_MODEL), lambda e, f: (0, 0)),
            pl.BlockSpec((T, N_EXPERTS), lambda e, f: (0, 0)),
            pl.BlockSpec((T, 1), lambda e, f: (0, 0)),
        ],
        out_shape=[
            jax.ShapeDtypeStruct((T, D_MODEL), jnp.float32),
            jax.ShapeDtypeStruct((T, N_EXPERTS), jnp.float32),
            jax.ShapeDtypeStruct((T, 1), jnp.float32),
        ],
        scratch_shapes=[
            pltpu.VMEM((T, N_EXPERTS), jnp.float32),
            pltpu.VMEM((T, N_EXPERTS), jnp.float32),
            pltpu.VMEM((SLOTS, N_EXPERTS), jnp.int32),
            pltpu.VMEM((SLOTS, N_EXPERTS), jnp.float32),
            pltpu.VMEM((SLOTS, D_MODEL), jnp.float32),
            pltpu.VMEM((SLOTS, D_MODEL), jnp.float32),
            pltpu.VMEM((T, 1), jnp.float32),
        ],
        compiler_params=pltpu.CompilerParams(
            dimension_semantics=("arbitrary", "arbitrary"),
        ),
        interpret=_INTERPRET,
    )(flat, prior, router_W, rb, b1r, b2r, W1, W2)

    return (out.reshape(B, T, D_MODEL),
            probs.reshape(B, T, N_EXPERTS),
            ent.reshape(B, T))


# speedup vs baseline: 8.0180x; 1.1696x over previous
import functools
import math

import jax
import jax.numpy as jnp
from jax.experimental import pallas as pl
from jax.experimental.pallas import tpu as pltpu

_INTERPRET = False

B = 1
T = 2048
D_MODEL = 768
ORGAN_DIM = 64
N_EXPERTS = 64
D_FF = 2048
CAPACITY = int(math.ceil(T * B / N_EXPERTS * 1.25))
SLOTS = 40
NFF = 1
FFB = D_FF // NFF
NEG_INF = float("-inf")


def _bf(x):
    return x.astype(jnp.bfloat16)


def _moe_kernel(flat_ref, prior_ref, rw_ref, rb_ref, b1_ref, b2_ref,
                w1_ref, w2_ref,
                out_ref, probs_ref, ent_ref,
                s_ref, wf_ref, sel_ref, selw_ref,
                xsel_ref, osel_ref, proc_ref):
    e = pl.program_id(0)
    f = pl.program_id(1)

    @pl.when((e == 0) & (f == 0))
    def _router():
        flat = flat_ref[:]
        inp = jnp.concatenate([flat, prior_ref[:]], axis=1)
        logits = jax.lax.dot_general(
            inp, rw_ref[:], (((1,), (1,)), ((), ())),
            preferred_element_type=jnp.float32) + rb_ref[:]
        m = jnp.max(logits, axis=1, keepdims=True)
        ex = jnp.exp(logits - m)
        probs = ex / jnp.sum(ex, axis=1, keepdims=True)
        probs_ref[:] = probs
        ent_ref[:] = -jnp.sum(probs * jnp.log(probs + 1e-12), axis=1,
                              keepdims=True)

        lane = jax.lax.broadcasted_iota(jnp.int32, (T, N_EXPERTS), 1)
        m1 = jnp.max(probs, axis=1, keepdims=True)
        i1 = jnp.min(jnp.where(probs == m1, lane, N_EXPERTS), axis=1,
                     keepdims=True)
        p2 = jnp.where(lane == i1, -1.0, probs)
        m2 = jnp.max(p2, axis=1, keepdims=True)
        i2 = jnp.min(jnp.where(p2 == m2, lane, N_EXPERTS), axis=1,
                     keepdims=True)
        denom = m1 + m2 + 1e-12
        hit1 = lane == i1
        hit2 = lane == i2
        wf_ref[:] = (jnp.where(hit1, m1 / denom, 0.0)
                     + jnp.where(hit2, m2 / denom, 0.0))
        s_ref[:] = jnp.where(hit1 | hit2, probs, NEG_INF)

        rows = jax.lax.broadcasted_iota(jnp.int32, (T, N_EXPERTS), 0)

        def body(k, _):
            sv = s_ref[:]
            mk = jnp.max(sv, axis=0, keepdims=True)
            cand = jnp.where(sv == mk, rows, T)
            idx = jnp.min(cand, axis=0, keepdims=True)
            valid = mk > -1e37
            picked = rows == idx
            wsel = jnp.sum(jnp.where(picked, wf_ref[:], 0.0), axis=0,
                           keepdims=True)
            sel_ref[pl.ds(k, 1), :] = jnp.where(valid, idx, T)
            selw_ref[pl.ds(k, 1), :] = jnp.where(valid, wsel, 0.0)
            s_ref[:] = jnp.where(picked, NEG_INF, sv)
            return 0

        jax.lax.fori_loop(0, CAPACITY, body, 0)

    lane = jax.lax.broadcasted_iota(jnp.int32, (SLOTS, N_EXPERTS), 1)
    sel_col = jnp.sum(jnp.where(lane == e, sel_ref[:], 0), axis=1,
                      keepdims=True)

    @pl.when(f == 0)
    def _gather():
        pt = (jax.lax.broadcasted_iota(jnp.int32, (SLOTS, T), 1)
              == sel_col).astype(jnp.bfloat16)
        xsel_ref[:] = jnp.dot(pt, _bf(flat_ref[:]),
                              preferred_element_type=jnp.float32)
        osel_ref[:] = jnp.zeros((SLOTS, D_MODEL), jnp.float32)

    a = jnp.dot(_bf(xsel_ref[:]), _bf(w1_ref[0]),
                preferred_element_type=jnp.float32) + b1_ref[0]
    h = 0.5 * a * (1.0 + jax.lax.erf(a * (1.0 / math.sqrt(2.0))))
    osel_ref[:] += jnp.dot(_bf(h), _bf(w2_ref[0]),
                           preferred_element_type=jnp.float32)

    @pl.when(f == NFF - 1)
    def _scatter():
        w_col = jnp.sum(jnp.where(lane == e, selw_ref[:], 0.0), axis=1,
                        keepdims=True)
        d0 = jax.lax.broadcasted_iota(jnp.int32, (SLOTS, SLOTS), 0)
        d1 = jax.lax.broadcasted_iota(jnp.int32, (SLOTS, SLOTS), 1)
        diag = d0 == d1
        sel_row = jnp.sum(jnp.where(diag, sel_col, 0), axis=0,
                          keepdims=True)
        w_row = jnp.sum(jnp.where(diag, w_col, 0.0), axis=0,
                        keepdims=True)
        tok = jax.lax.broadcasted_iota(jnp.int32, (T, SLOTS), 0)
        p01 = tok == sel_row
        pw = jnp.where(p01, w_row, 0.0).astype(jnp.bfloat16)

        @pl.when(e == 0)
        def _():
            out_ref[:] = jnp.zeros((T, D_MODEL), jnp.float32)
            proc_ref[:] = jnp.zeros((T, 1), jnp.float32)

        osel = osel_ref[:] + b2_ref[0]
        out_ref[:] += jnp.dot(pw, _bf(osel),
                              preferred_element_type=jnp.float32)
        proc_ref[:] += jnp.sum(p01.astype(jnp.float32), axis=1,
                               keepdims=True)

        @pl.when(e == N_EXPERTS - 1)
        def _():
            out_ref[:] = jnp.where(proc_ref[:] > 0.0, out_ref[:],
                                   flat_ref[:])


@jax.jit
def kernel(tokens, organ_priors, router_W, router_b, W1, b1, W2, b2):
    flat = tokens.reshape(T, D_MODEL)
    prior = organ_priors.reshape(T, ORGAN_DIM)
    rb = router_b.reshape(1, N_EXPERTS)
    b1r = b1.reshape(N_EXPERTS, 1, D_FF)
    b2r = b2.reshape(N_EXPERTS, 1, D_MODEL)

    out, probs, ent = pl.pallas_call(
        _moe_kernel,
        grid=(N_EXPERTS, NFF),
        in_specs=[
            pl.BlockSpec((T, D_MODEL), lambda e, f: (0, 0)),
            pl.BlockSpec((T, ORGAN_DIM), lambda e, f: (0, 0)),
            pl.BlockSpec((N_EXPERTS, D_MODEL + ORGAN_DIM),
                         lambda e, f: (0, 0)),
            pl.BlockSpec((1, N_EXPERTS), lambda e, f: (0, 0)),
            pl.BlockSpec((1, 1, FFB), lambda e, f: (e, 0, f)),
            pl.BlockSpec((1, 1, D_MODEL), lambda e, f: (e, 0, 0)),
            pl.BlockSpec((1, D_MODEL, FFB), lambda e, f: (e, 0, f)),
            pl.BlockSpec((1, FFB, D_MODEL), lambda e, f: (e, f, 0)),
        ],
        out_specs=[
            pl.BlockSpec((T, D_MODEL), lambda e, f: (0, 0)),
            pl.BlockSpec((T, N_EXPERTS), lambda e, f: (0, 0)),
            pl.BlockSpec((T, 1), lambda e, f: (0, 0)),
        ],
        out_shape=[
            jax.ShapeDtypeStruct((T, D_MODEL), jnp.float32),
            jax.ShapeDtypeStruct((T, N_EXPERTS), jnp.float32),
            jax.ShapeDtypeStruct((T, 1), jnp.float32),
        ],
        scratch_shapes=[
            pltpu.VMEM((T, N_EXPERTS), jnp.float32),
            pltpu.VMEM((T, N_EXPERTS), jnp.float32),
            pltpu.VMEM((SLOTS, N_EXPERTS), jnp.int32),
            pltpu.VMEM((SLOTS, N_EXPERTS), jnp.float32),
            pltpu.VMEM((SLOTS, D_MODEL), jnp.float32),
            pltpu.VMEM((SLOTS, D_MODEL), jnp.float32),
            pltpu.VMEM((T, 1), jnp.float32),
        ],
        compiler_params=pltpu.CompilerParams(
            dimension_semantics=("arbitrary", "arbitrary"),
        ),
        interpret=_INTERPRET,
    )(flat, prior, router_W, rb, b1r, b2r, W1, W2)

    return (out.reshape(B, T, D_MODEL),
            probs.reshape(B, T, N_EXPERTS),
            ent.reshape(B, T))


# speedup vs baseline: 8.1422x; 1.0155x over previous
import functools
import math

import jax
import jax.numpy as jnp
from jax.experimental import pallas as pl
from jax.experimental.pallas import tpu as pltpu

_INTERPRET = False

B = 1
T = 2048
D_MODEL = 768
ORGAN_DIM = 64
N_EXPERTS = 64
D_FF = 2048
CAPACITY = int(math.ceil(T * B / N_EXPERTS * 1.25))
SLOTS = 40
G = 8
GS = G * SLOTS
NEG_INF = float("-inf")


def _bf(x):
    return x.astype(jnp.bfloat16)


def _moe_kernel(flat_ref, prior_ref, rw_ref, rb_ref, b1_ref, b2_ref,
                w1_ref, w2_ref,
                out_ref, probs_ref, ent_ref,
                s_ref, wf_ref, sel_ref, selw_ref,
                xsel_ref, osel_ref, wslot_ref, proc_ref):
    e = pl.program_id(0)
    g = jax.lax.rem(e, G)
    e0 = e - g

    @pl.when(e == 0)
    def _router():
        flat = flat_ref[:]
        inp = jnp.concatenate([flat, prior_ref[:]], axis=1)
        logits = jax.lax.dot_general(
            inp, rw_ref[:], (((1,), (1,)), ((), ())),
            preferred_element_type=jnp.float32) + rb_ref[:]
        m = jnp.max(logits, axis=1, keepdims=True)
        ex = jnp.exp(logits - m)
        probs = ex / jnp.sum(ex, axis=1, keepdims=True)
        probs_ref[:] = probs
        ent_ref[:] = -jnp.sum(probs * jnp.log(probs + 1e-12), axis=1,
                              keepdims=True)

        lane = jax.lax.broadcasted_iota(jnp.int32, (T, N_EXPERTS), 1)
        m1 = jnp.max(probs, axis=1, keepdims=True)
        i1 = jnp.min(jnp.where(probs == m1, lane, N_EXPERTS), axis=1,
                     keepdims=True)
        p2 = jnp.where(lane == i1, -1.0, probs)
        m2 = jnp.max(p2, axis=1, keepdims=True)
        i2 = jnp.min(jnp.where(p2 == m2, lane, N_EXPERTS), axis=1,
                     keepdims=True)
        denom = m1 + m2 + 1e-12
        hit1 = lane == i1
        hit2 = lane == i2
        wf_ref[:] = (jnp.where(hit1, m1 / denom, 0.0)
                     + jnp.where(hit2, m2 / denom, 0.0))
        s_ref[:] = jnp.where(hit1 | hit2, probs, NEG_INF)

        rows = jax.lax.broadcasted_iota(jnp.int32, (T, N_EXPERTS), 0)

        def body(k, _):
            sv = s_ref[:]
            mk = jnp.max(sv, axis=0, keepdims=True)
            cand = jnp.where(sv == mk, rows, T)
            idx = jnp.min(cand, axis=0, keepdims=True)
            valid = mk > -1e37
            picked = rows == idx
            wsel = jnp.sum(jnp.where(picked, wf_ref[:], 0.0), axis=0,
                           keepdims=True)
            sel_ref[pl.ds(k, 1), :] = jnp.where(valid, idx, T)
            selw_ref[pl.ds(k, 1), :] = jnp.where(valid, wsel, 0.0)
            s_ref[:] = jnp.where(picked, NEG_INF, sv)
            return 0

        jax.lax.fori_loop(0, CAPACITY, body, 0)

        proc_ref[:] = jnp.zeros((T, 1), jnp.float32)
        out_ref[:] = jnp.zeros((T, D_MODEL), jnp.float32)

    lane_s = jax.lax.broadcasted_iota(jnp.int32, (SLOTS, N_EXPERTS), 1)

    @pl.when(g == 0)
    def _gather():
        for gg in range(G):
            sc = jnp.sum(jnp.where(lane_s == e0 + gg, sel_ref[:], 0),
                         axis=1, keepdims=True)
            wc = jnp.sum(jnp.where(lane_s == e0 + gg, selw_ref[:], 0.0),
                         axis=1, keepdims=True)
            wslot_ref[pl.ds(gg * SLOTS, SLOTS), pl.ds(0, 1)] = sc.astype(
                jnp.float32)
            wslot_ref[pl.ds(gg * SLOTS, SLOTS), pl.ds(1, 1)] = wc
        selc = wslot_ref[:, pl.ds(0, 1)].astype(jnp.int32)
        p8t = (jax.lax.broadcasted_iota(jnp.int32, (GS, T), 1)
               == selc).astype(jnp.bfloat16)
        xsel_ref[:] = jnp.dot(p8t, _bf(flat_ref[:]),
                              preferred_element_type=jnp.float32)

    a = jnp.dot(_bf(xsel_ref[pl.ds(g * SLOTS, SLOTS), :]), _bf(w1_ref[0]),
                preferred_element_type=jnp.float32) + b1_ref[0]
    h = 0.5 * a * (1.0 + jax.lax.erf(a * (1.0 / math.sqrt(2.0))))
    osel_ref[pl.ds(g * SLOTS, SLOTS), :] = jnp.dot(
        _bf(h), _bf(w2_ref[0]),
        preferred_element_type=jnp.float32) + b2_ref[0]

    @pl.when(g == G - 1)
    def _scatter():
        selc = wslot_ref[:, pl.ds(0, 1)].astype(jnp.int32)
        wcol = wslot_ref[:, pl.ds(1, 1)]
        d0 = jax.lax.broadcasted_iota(jnp.int32, (GS, GS), 0)
        d1 = jax.lax.broadcasted_iota(jnp.int32, (GS, GS), 1)
        diag = d0 == d1
        sel_row = jnp.sum(jnp.where(diag, selc, 0), axis=0,
                          keepdims=True)
        tok = jax.lax.broadcasted_iota(jnp.int32, (T, GS), 0)
        p8 = (tok == sel_row).astype(jnp.bfloat16)
        osel_w = _bf(osel_ref[:] * wcol)
        out_ref[:] += jnp.dot(p8, osel_w,
                              preferred_element_type=jnp.float32)
        proc_ref[:] += jnp.dot(p8, jnp.ones((GS, 1), jnp.bfloat16),
                               preferred_element_type=jnp.float32)

        @pl.when(e == N_EXPERTS - 1)
        def _():
            out_ref[:] = jnp.where(proc_ref[:] > 0.0, out_ref[:],
                                   flat_ref[:])


@jax.jit
def kernel(tokens, organ_priors, router_W, router_b, W1, b1, W2, b2):
    flat = tokens.reshape(T, D_MODEL)
    prior = organ_priors.reshape(T, ORGAN_DIM)
    rb = router_b.reshape(1, N_EXPERTS)
    b1r = b1.reshape(N_EXPERTS, 1, D_FF)
    b2r = b2.reshape(N_EXPERTS, 1, D_MODEL)

    out, probs, ent = pl.pallas_call(
        _moe_kernel,
        grid=(N_EXPERTS,),
        in_specs=[
            pl.BlockSpec((T, D_MODEL), lambda e: (0, 0)),
            pl.BlockSpec((T, ORGAN_DIM), lambda e: (0, 0)),
            pl.BlockSpec((N_EXPERTS, D_MODEL + ORGAN_DIM),
                         lambda e: (0, 0)),
            pl.BlockSpec((1, N_EXPERTS), lambda e: (0, 0)),
            pl.BlockSpec((1, 1, D_FF), lambda e: (e, 0, 0)),
            pl.BlockSpec((1, 1, D_MODEL), lambda e: (e, 0, 0)),
            pl.BlockSpec((1, D_MODEL, D_FF), lambda e: (e, 0, 0)),
            pl.BlockSpec((1, D_FF, D_MODEL), lambda e: (e, 0, 0)),
        ],
        out_specs=[
            pl.BlockSpec((T, D_MODEL), lambda e: (0, 0)),
            pl.BlockSpec((T, N_EXPERTS), lambda e: (0, 0)),
            pl.BlockSpec((T, 1), lambda e: (0, 0)),
        ],
        out_shape=[
            jax.ShapeDtypeStruct((T, D_MODEL), jnp.float32),
            jax.ShapeDtypeStruct((T, N_EXPERTS), jnp.float32),
            jax.ShapeDtypeStruct((T, 1), jnp.float32),
        ],
        scratch_shapes=[
            pltpu.VMEM((T, N_EXPERTS), jnp.float32),
            pltpu.VMEM((T, N_EXPERTS), jnp.float32),
            pltpu.VMEM((SLOTS, N_EXPERTS), jnp.int32),
            pltpu.VMEM((SLOTS, N_EXPERTS), jnp.float32),
            pltpu.VMEM((GS, D_MODEL), jnp.float32),
            pltpu.VMEM((GS, D_MODEL), jnp.float32),
            pltpu.VMEM((GS, 8), jnp.float32),
            pltpu.VMEM((T, 1), jnp.float32),
        ],
        compiler_params=pltpu.CompilerParams(
            dimension_semantics=("arbitrary",),
        ),
        interpret=_INTERPRET,
    )(flat, prior, router_W, rb, b1r, b2r, W1, W2)

    return (out.reshape(B, T, D_MODEL),
            probs.reshape(B, T, N_EXPERTS),
            ent.reshape(B, T))


# speedup vs baseline: 8.3560x; 1.0262x over previous
import functools
import math

import jax
import jax.numpy as jnp
from jax.experimental import pallas as pl
from jax.experimental.pallas import tpu as pltpu

_INTERPRET = False

B = 1
T = 2048
D_MODEL = 768
ORGAN_DIM = 64
N_EXPERTS = 64
D_FF = 2048
CAPACITY = int(math.ceil(T * B / N_EXPERTS * 1.25))
SLOTS = 40
G = 8
GS = G * SLOTS
NEG_INF = float("-inf")


def _bf(x):
    return x.astype(jnp.bfloat16)


def _moe_kernel(flat_ref, prior_ref, rw_ref, rb_ref, b1_ref, b2_ref,
                w1_ref, w2_ref,
                out_ref, probs_ref, ent_ref,
                s_ref, wf_ref, sel_ref, selw_ref,
                xsel_ref, osel_ref, wslot_ref, proc_ref):
    e = pl.program_id(0)
    g = jax.lax.rem(e, G)
    e0 = e - g

    @pl.when(e == 0)
    def _router():
        flat = flat_ref[:]
        inp = jnp.concatenate([flat, prior_ref[:]], axis=1)
        logits = jax.lax.dot_general(
            inp, rw_ref[:], (((1,), (1,)), ((), ())),
            preferred_element_type=jnp.float32) + rb_ref[:]
        m = jnp.max(logits, axis=1, keepdims=True)
        ex = jnp.exp(logits - m)
        probs = ex / jnp.sum(ex, axis=1, keepdims=True)
        probs_ref[:] = probs
        ent_ref[:] = -jnp.sum(probs * jnp.log(probs + 1e-12), axis=1,
                              keepdims=True)

        lane = jax.lax.broadcasted_iota(jnp.int32, (T, N_EXPERTS), 1)
        m1 = jnp.max(probs, axis=1, keepdims=True)
        i1 = jnp.min(jnp.where(probs == m1, lane, N_EXPERTS), axis=1,
                     keepdims=True)
        p2 = jnp.where(lane == i1, -1.0, probs)
        m2 = jnp.max(p2, axis=1, keepdims=True)
        i2 = jnp.min(jnp.where(p2 == m2, lane, N_EXPERTS), axis=1,
                     keepdims=True)
        denom = m1 + m2 + 1e-12
        hit1 = lane == i1
        hit2 = lane == i2
        wf = (jnp.where(hit1, m1 / denom, 0.0)
              + jnp.where(hit2, m2 / denom, 0.0))
        s = jnp.where(hit1 | hit2, probs, NEG_INF)

        TH = T // 2
        s_ref[:] = jnp.concatenate([s[:TH], s[TH:]], axis=1)
        wf_ref[:] = jnp.concatenate([wf[:TH], wf[TH:]], axis=1)

        half = jax.lax.broadcasted_iota(jnp.int32, (TH, 2 * N_EXPERTS), 1)
        rowsg = (jax.lax.broadcasted_iota(jnp.int32, (TH, 2 * N_EXPERTS), 0)
                 + jnp.where(half >= N_EXPERTS, TH, 0))

        def body(k, _):
            sv = s_ref[:]
            mh = jnp.max(sv, axis=0, keepdims=True)
            m64 = jnp.maximum(mh[:, :N_EXPERTS], mh[:, N_EXPERTS:])
            mkb = jnp.concatenate([m64, m64], axis=1)
            cand = jnp.where(sv == mkb, rowsg, T)
            ih = jnp.min(cand, axis=0, keepdims=True)
            i64 = jnp.minimum(ih[:, :N_EXPERTS], ih[:, N_EXPERTS:])
            idxb = jnp.concatenate([i64, i64], axis=1)
            valid = m64 > -1e37
            picked = rowsg == idxb
            wh = jnp.sum(jnp.where(picked, wf_ref[:], 0.0), axis=0,
                         keepdims=True)
            wsel = wh[:, :N_EXPERTS] + wh[:, N_EXPERTS:]
            sel_ref[pl.ds(k, 1), :] = jnp.where(valid, i64, T)
            selw_ref[pl.ds(k, 1), :] = jnp.where(valid, wsel, 0.0)
            s_ref[:] = jnp.where(picked, NEG_INF, sv)
            return 0

        jax.lax.fori_loop(0, CAPACITY, body, 0)

        proc_ref[:] = jnp.zeros((T, 1), jnp.float32)
        out_ref[:] = jnp.zeros((T, D_MODEL), jnp.float32)

    lane_s = jax.lax.broadcasted_iota(jnp.int32, (SLOTS, N_EXPERTS), 1)

    @pl.when(g == 0)
    def _gather():
        for gg in range(G):
            sc = jnp.sum(jnp.where(lane_s == e0 + gg, sel_ref[:], 0),
                         axis=1, keepdims=True)
            wc = jnp.sum(jnp.where(lane_s == e0 + gg, selw_ref[:], 0.0),
                         axis=1, keepdims=True)
            wslot_ref[pl.ds(gg * SLOTS, SLOTS), pl.ds(0, 1)] = sc.astype(
                jnp.float32)
            wslot_ref[pl.ds(gg * SLOTS, SLOTS), pl.ds(1, 1)] = wc
        selc = wslot_ref[:, pl.ds(0, 1)].astype(jnp.int32)
        p8t = (jax.lax.broadcasted_iota(jnp.int32, (GS, T), 1)
               == selc).astype(jnp.bfloat16)
        xsel_ref[:] = jnp.dot(p8t, _bf(flat_ref[:]),
                              preferred_element_type=jnp.float32)

    a = jnp.dot(_bf(xsel_ref[pl.ds(g * SLOTS, SLOTS), :]), _bf(w1_ref[0]),
                preferred_element_type=jnp.float32) + b1_ref[0]
    h = 0.5 * a * (1.0 + jax.lax.erf(a * (1.0 / math.sqrt(2.0))))
    osel_ref[pl.ds(g * SLOTS, SLOTS), :] = jnp.dot(
        _bf(h), _bf(w2_ref[0]),
        preferred_element_type=jnp.float32) + b2_ref[0]

    @pl.when(g == G - 1)
    def _scatter():
        selc = wslot_ref[:, pl.ds(0, 1)].astype(jnp.int32)
        wcol = wslot_ref[:, pl.ds(1, 1)]
        d0 = jax.lax.broadcasted_iota(jnp.int32, (GS, GS), 0)
        d1 = jax.lax.broadcasted_iota(jnp.int32, (GS, GS), 1)
        diag = d0 == d1
        sel_row = jnp.sum(jnp.where(diag, selc, 0), axis=0,
                          keepdims=True)
        tok = jax.lax.broadcasted_iota(jnp.int32, (T, GS), 0)
        p8 = (tok == sel_row).astype(jnp.bfloat16)
        osel_w = _bf(osel_ref[:] * wcol)
        out_ref[:] += jnp.dot(p8, osel_w,
                              preferred_element_type=jnp.float32)
        proc_ref[:] += jnp.dot(p8, jnp.ones((GS, 1), jnp.bfloat16),
                               preferred_element_type=jnp.float32)

        @pl.when(e == N_EXPERTS - 1)
        def _():
            out_ref[:] = jnp.where(proc_ref[:] > 0.0, out_ref[:],
                                   flat_ref[:])


@jax.jit
def kernel(tokens, organ_priors, router_W, router_b, W1, b1, W2, b2):
    flat = tokens.reshape(T, D_MODEL)
    prior = organ_priors.reshape(T, ORGAN_DIM)
    rb = router_b.reshape(1, N_EXPERTS)
    b1r = b1.reshape(N_EXPERTS, 1, D_FF)
    b2r = b2.reshape(N_EXPERTS, 1, D_MODEL)

    out, probs, ent = pl.pallas_call(
        _moe_kernel,
        grid=(N_EXPERTS,),
        in_specs=[
            pl.BlockSpec((T, D_MODEL), lambda e: (0, 0)),
            pl.BlockSpec((T, ORGAN_DIM), lambda e: (0, 0)),
            pl.BlockSpec((N_EXPERTS, D_MODEL + ORGAN_DIM),
                         lambda e: (0, 0)),
            pl.BlockSpec((1, N_EXPERTS), lambda e: (0, 0)),
            pl.BlockSpec((1, 1, D_FF), lambda e: (e, 0, 0)),
            pl.BlockSpec((1, 1, D_MODEL), lambda e: (e, 0, 0)),
            pl.BlockSpec((1, D_MODEL, D_FF), lambda e: (e, 0, 0)),
            pl.BlockSpec((1, D_FF, D_MODEL), lambda e: (e, 0, 0)),
        ],
        out_specs=[
            pl.BlockSpec((T, D_MODEL), lambda e: (0, 0)),
            pl.BlockSpec((T, N_EXPERTS), lambda e: (0, 0)),
            pl.BlockSpec((T, 1), lambda e: (0, 0)),
        ],
        out_shape=[
            jax.ShapeDtypeStruct((T, D_MODEL), jnp.float32),
            jax.ShapeDtypeStruct((T, N_EXPERTS), jnp.float32),
            jax.ShapeDtypeStruct((T, 1), jnp.float32),
        ],
        scratch_shapes=[
            pltpu.VMEM((T // 2, 2 * N_EXPERTS), jnp.float32),
            pltpu.VMEM((T // 2, 2 * N_EXPERTS), jnp.float32),
            pltpu.VMEM((SLOTS, N_EXPERTS), jnp.int32),
            pltpu.VMEM((SLOTS, N_EXPERTS), jnp.float32),
            pltpu.VMEM((GS, D_MODEL), jnp.float32),
            pltpu.VMEM((GS, D_MODEL), jnp.float32),
            pltpu.VMEM((GS, 8), jnp.float32),
            pltpu.VMEM((T, 1), jnp.float32),
        ],
        compiler_params=pltpu.CompilerParams(
            dimension_semantics=("arbitrary",),
        ),
        interpret=_INTERPRET,
    )(flat, prior, router_W, rb, b1r, b2r, W1, W2)

    return (out.reshape(B, T, D_MODEL),
            probs.reshape(B, T, N_EXPERTS),
            ent.reshape(B, T))


# speedup vs baseline: 8.6543x; 1.0357x over previous
import functools
import math

import jax
import jax.numpy as jnp
from jax.experimental import pallas as pl
from jax.experimental.pallas import tpu as pltpu

_INTERPRET = False

B = 1
T = 2048
D_MODEL = 768
ORGAN_DIM = 64
N_EXPERTS = 64
D_FF = 2048
CAPACITY = int(math.ceil(T * B / N_EXPERTS * 1.25))
SLOTS = 40
G = 8
GS = G * SLOTS
NEG_INF = float("-inf")


def _bf(x):
    return x.astype(jnp.bfloat16)


def _moe_kernel(flat_ref, prior_ref, rw_ref, rb_ref, b1_ref, b2_ref,
                w1_ref, w2_ref,
                out_ref, probs_ref, ent_ref,
                s_ref, wf_ref, sel_ref,
                xsel_ref, osel_ref, wslot_ref, proc_ref):
    e = pl.program_id(0)
    g = jax.lax.rem(e, G)
    e0 = e - g

    @pl.when(e == 0)
    def _router():
        flat = flat_ref[:]
        inp = jnp.concatenate([flat, prior_ref[:]], axis=1)
        logits = jax.lax.dot_general(
            inp, rw_ref[:], (((1,), (1,)), ((), ())),
            preferred_element_type=jnp.float32) + rb_ref[:]
        m = jnp.max(logits, axis=1, keepdims=True)
        ex = jnp.exp(logits - m)
        probs = ex / jnp.sum(ex, axis=1, keepdims=True)
        probs_ref[:] = probs
        ent_ref[:] = -jnp.sum(probs * jnp.log(probs + 1e-12), axis=1,
                              keepdims=True)

        lane = jax.lax.broadcasted_iota(jnp.int32, (T, N_EXPERTS), 1)
        m1 = jnp.max(probs, axis=1, keepdims=True)
        i1 = jnp.min(jnp.where(probs == m1, lane, N_EXPERTS), axis=1,
                     keepdims=True)
        p2 = jnp.where(lane == i1, -1.0, probs)
        m2 = jnp.max(p2, axis=1, keepdims=True)
        i2 = jnp.min(jnp.where(p2 == m2, lane, N_EXPERTS), axis=1,
                     keepdims=True)
        denom = m1 + m2 + 1e-12
        hit1 = lane == i1
        hit2 = lane == i2
        wf = (jnp.where(hit1, m1 / denom, 0.0)
              + jnp.where(hit2, m2 / denom, 0.0))
        wf_ref[:] = wf.astype(jnp.bfloat16)
        s = jnp.where(hit1 | hit2, probs, NEG_INF)

        TH = T // 2
        s_ref[:] = jnp.concatenate([s[:TH], s[TH:]], axis=1)

        half = jax.lax.broadcasted_iota(jnp.int32, (TH, 2 * N_EXPERTS), 1)
        rowsg = (jax.lax.broadcasted_iota(jnp.int32, (TH, 2 * N_EXPERTS), 0)
                 + jnp.where(half >= N_EXPERTS, TH, 0))

        def body(k, _):
            sv = s_ref[:]
            mh = jnp.max(sv, axis=0, keepdims=True)
            m64 = jnp.maximum(mh[:, :N_EXPERTS], mh[:, N_EXPERTS:])
            mkb = jnp.concatenate([m64, m64], axis=1)
            cand = jnp.where(sv == mkb, rowsg, T)
            ih = jnp.min(cand, axis=0, keepdims=True)
            i64 = jnp.minimum(ih[:, :N_EXPERTS], ih[:, N_EXPERTS:])
            idxb = jnp.concatenate([i64, i64], axis=1)
            valid = m64 > -1e37
            picked = rowsg == idxb
            sel_ref[pl.ds(k, 1), :] = jnp.where(valid, i64, T)
            s_ref[:] = jnp.where(picked, NEG_INF, sv)
            return 0

        jax.lax.fori_loop(0, CAPACITY, body, 0)

        proc_ref[:] = jnp.zeros((T, 1), jnp.float32)
        out_ref[:] = jnp.zeros((T, D_MODEL), jnp.float32)

    lane_s = jax.lax.broadcasted_iota(jnp.int32, (SLOTS, N_EXPERTS), 1)

    @pl.when(g == 0)
    def _gather():
        for gg in range(G):
            sc = jnp.sum(jnp.where(lane_s == e0 + gg, sel_ref[:], 0),
                         axis=1, keepdims=True)
            wslot_ref[pl.ds(gg * SLOTS, SLOTS), pl.ds(0, 1)] = sc.astype(
                jnp.float32)
        selc = wslot_ref[:, pl.ds(0, 1)].astype(jnp.int32)
        p8t = (jax.lax.broadcasted_iota(jnp.int32, (GS, T), 1)
               == selc).astype(jnp.bfloat16)
        xsel_ref[:] = jnp.dot(p8t, _bf(flat_ref[:]),
                              preferred_element_type=jnp.float32)

    @pl.when(g == 1)
    def _weights():
        selc = wslot_ref[:, pl.ds(0, 1)].astype(jnp.int32)
        p8t = (jax.lax.broadcasted_iota(jnp.int32, (GS, T), 1)
               == selc).astype(jnp.bfloat16)
        w_all = jnp.dot(p8t, wf_ref[:],
                        preferred_element_type=jnp.float32)
        lane_w = jax.lax.broadcasted_iota(jnp.int32, (GS, N_EXPERTS), 1)
        for gg in range(G):
            wcol = jnp.sum(jnp.where(lane_w == e0 + gg, w_all, 0.0),
                           axis=1, keepdims=True)
            wslot_ref[pl.ds(gg * SLOTS, SLOTS), pl.ds(1, 1)] = (
                wcol[gg * SLOTS:(gg + 1) * SLOTS])

    a = jnp.dot(_bf(xsel_ref[pl.ds(g * SLOTS, SLOTS), :]), _bf(w1_ref[0]),
                preferred_element_type=jnp.float32) + b1_ref[0]
    h = 0.5 * a * (1.0 + jax.lax.erf(a * (1.0 / math.sqrt(2.0))))
    osel_ref[pl.ds(g * SLOTS, SLOTS), :] = jnp.dot(
        _bf(h), _bf(w2_ref[0]),
        preferred_element_type=jnp.float32) + b2_ref[0]

    @pl.when(g == G - 1)
    def _scatter():
        selc = wslot_ref[:, pl.ds(0, 1)].astype(jnp.int32)
        wcol = wslot_ref[:, pl.ds(1, 1)]
        d0 = jax.lax.broadcasted_iota(jnp.int32, (GS, GS), 0)
        d1 = jax.lax.broadcasted_iota(jnp.int32, (GS, GS), 1)
        diag = d0 == d1
        sel_row = jnp.sum(jnp.where(diag, selc, 0), axis=0,
                          keepdims=True)
        tok = jax.lax.broadcasted_iota(jnp.int32, (T, GS), 0)
        p8 = (tok == sel_row).astype(jnp.bfloat16)
        osel_w = _bf(osel_ref[:] * wcol)
        out_ref[:] += jnp.dot(p8, osel_w,
                              preferred_element_type=jnp.float32)
        proc_ref[:] += jnp.dot(p8, jnp.ones((GS, 1), jnp.bfloat16),
                               preferred_element_type=jnp.float32)

        @pl.when(e == N_EXPERTS - 1)
        def _():
            out_ref[:] = jnp.where(proc_ref[:] > 0.0, out_ref[:],
                                   flat_ref[:])


@jax.jit
def kernel(tokens, organ_priors, router_W, router_b, W1, b1, W2, b2):
    flat = tokens.reshape(T, D_MODEL)
    prior = organ_priors.reshape(T, ORGAN_DIM)
    rb = router_b.reshape(1, N_EXPERTS)
    b1r = b1.reshape(N_EXPERTS, 1, D_FF)
    b2r = b2.reshape(N_EXPERTS, 1, D_MODEL)

    out, probs, ent = pl.pallas_call(
        _moe_kernel,
        grid=(N_EXPERTS,),
        in_specs=[
            pl.BlockSpec((T, D_MODEL), lambda e: (0, 0)),
            pl.BlockSpec((T, ORGAN_DIM), lambda e: (0, 0)),
            pl.BlockSpec((N_EXPERTS, D_MODEL + ORGAN_DIM),
                         lambda e: (0, 0)),
            pl.BlockSpec((1, N_EXPERTS), lambda e: (0, 0)),
            pl.BlockSpec((1, 1, D_FF), lambda e: (e, 0, 0)),
            pl.BlockSpec((1, 1, D_MODEL), lambda e: (e, 0, 0)),
            pl.BlockSpec((1, D_MODEL, D_FF), lambda e: (e, 0, 0)),
            pl.BlockSpec((1, D_FF, D_MODEL), lambda e: (e, 0, 0)),
        ],
        out_specs=[
            pl.BlockSpec((T, D_MODEL), lambda e: (0, 0)),
            pl.BlockSpec((T, N_EXPERTS), lambda e: (0, 0)),
            pl.BlockSpec((T, 1), lambda e: (0, 0)),
        ],
        out_shape=[
            jax.ShapeDtypeStruct((T, D_MODEL), jnp.float32),
            jax.ShapeDtypeStruct((T, N_EXPERTS), jnp.float32),
            jax.ShapeDtypeStruct((T, 1), jnp.float32),
        ],
        scratch_shapes=[
            pltpu.VMEM((T // 2, 2 * N_EXPERTS), jnp.float32),
            pltpu.VMEM((T, N_EXPERTS), jnp.bfloat16),
            pltpu.VMEM((SLOTS, N_EXPERTS), jnp.int32),
            pltpu.VMEM((GS, D_MODEL), jnp.float32),
            pltpu.VMEM((GS, D_MODEL), jnp.float32),
            pltpu.VMEM((GS, 8), jnp.float32),
            pltpu.VMEM((T, 1), jnp.float32),
        ],
        compiler_params=pltpu.CompilerParams(
            dimension_semantics=("arbitrary",),
        ),
        interpret=_INTERPRET,
    )(flat, prior, router_W, rb, b1r, b2r, W1, W2)

    return (out.reshape(B, T, D_MODEL),
            probs.reshape(B, T, N_EXPERTS),
            ent.reshape(B, T))
